# baseline (speedup 1.0000x reference)
import numpy as np

import jax
import jax.numpy as jnp
from jax import lax
from jax.experimental import pallas as pl
from jax.sharding import Mesh, PartitionSpec as P

try:
    from jax import shard_map as _shard_map_impl

    def _shard_map(f, *, mesh, in_specs, out_specs):
        return _shard_map_impl(f, mesh=mesh, in_specs=in_specs,
                               out_specs=out_specs, check_vma=False)
except ImportError:
    from jax.experimental.shard_map import shard_map as _shard_map_impl

    def _shard_map(f, *, mesh, in_specs, out_specs):
        return _shard_map_impl(f, mesh=mesh, in_specs=in_specs,
                               out_specs=out_specs, check_rep=False)

_HIDDEN = 4096
_RH = 1024
_E = 16
_TB = 256


def _router_kernel(x_ref, w1t_ref, b1_ref, w2t_ref, b2_ref,
                   w_ref, tkw_ref, tki_ref, logits_ref):
    x = x_ref[...].astype(jnp.bfloat16)
    h = jnp.dot(x, w1t_ref[...],
                preferred_element_type=jnp.float32) + b1_ref[...]
    g = 0.5 * h * (1.0 + lax.erf(h * 0.7071067811865476))
    logits = jnp.dot(g.astype(jnp.bfloat16), w2t_ref[...],
                     preferred_element_type=jnp.float32) + b2_ref[...]
    logits_ref[...] = logits

    m = jnp.max(logits, axis=-1, keepdims=True)
    e = jnp.exp(logits - m)
    s = jnp.sum(e, axis=-1, keepdims=True)
    w = e / s
    w_ref[...] = w

    iota = lax.broadcasted_iota(jnp.int32, w.shape, 1)
    m1 = jnp.max(w, axis=-1, keepdims=True)
    i1 = jnp.min(jnp.where(w == m1, iota, _E), axis=-1, keepdims=True)
    wm = jnp.where(iota == i1, -1.0, w)
    m2 = jnp.max(wm, axis=-1, keepdims=True)
    i2 = jnp.min(jnp.where(wm == m2, iota, _E), axis=-1, keepdims=True)
    s2 = m1 + m2 + 1e-10
    tkw_ref[...] = jnp.concatenate([m1 / s2, m2 / s2], axis=1)
    tki_ref[...] = jnp.concatenate([i1, i2], axis=1)


def _router_call(instruction_features, w1t, b1r, w2t, b2r):
    B = instruction_features.shape[0]
    grid = (B // _TB,)
    out = pl.pallas_call(
        _router_kernel,
        grid=grid,
        in_specs=[
            pl.BlockSpec((_TB, _HIDDEN), lambda i: (i, 0)),
            pl.BlockSpec((_HIDDEN, _RH), lambda i: (0, 0)),
            pl.BlockSpec((1, _RH), lambda i: (0, 0)),
            pl.BlockSpec((_RH, _E), lambda i: (0, 0)),
            pl.BlockSpec((1, _E), lambda i: (0, 0)),
        ],
        out_specs=[
            pl.BlockSpec((_TB, _E), lambda i: (i, 0)),
            pl.BlockSpec((_TB, 2), lambda i: (i, 0)),
            pl.BlockSpec((_TB, 2), lambda i: (i, 0)),
            pl.BlockSpec((_TB, _E), lambda i: (i, 0)),
        ],
        out_shape=[
            jax.ShapeDtypeStruct((B, _E), jnp.float32),
            jax.ShapeDtypeStruct((B, 2), jnp.float32),
            jax.ShapeDtypeStruct((B, 2), jnp.int32),
            jax.ShapeDtypeStruct((B, _E), jnp.float32),
        ],
    )(instruction_features, w1t, b1r, w2t, b2r)
    routing_weights, top_k_weights, top_k_indices, router_logits = out
    return (routing_weights, top_k_weights, top_k_indices, router_logits)


def kernel(instruction_features, W1, b1, W2, b2):
    w1t = W1.T.astype(jnp.bfloat16)
    w2t = W2.T.astype(jnp.bfloat16)
    b1r = b1.reshape(1, _RH)
    b2r = b2.reshape(1, _E)
    return _router_call(instruction_features, w1t, b1r, w2t, b2r)


# speedup vs baseline: 1.1405x; 1.1405x over previous
import numpy as np

import jax
import jax.numpy as jnp
from jax import lax
from jax.experimental import pallas as pl
from jax.sharding import Mesh, PartitionSpec as P

try:
    from jax import shard_map as _shard_map_impl

    def _shard_map(f, *, mesh, in_specs, out_specs):
        return _shard_map_impl(f, mesh=mesh, in_specs=in_specs,
                               out_specs=out_specs, check_vma=False)
except ImportError:
    from jax.experimental.shard_map import shard_map as _shard_map_impl

    def _shard_map(f, *, mesh, in_specs, out_specs):
        return _shard_map_impl(f, mesh=mesh, in_specs=in_specs,
                               out_specs=out_specs, check_rep=False)

_HIDDEN = 4096
_RH = 1024
_E = 16
_TB = 1024


def _router_kernel(x_ref, w1t_ref, b1_ref, w2t_ref, b2_ref,
                   w_ref, tkw_ref, tki_ref, logits_ref):
    x = x_ref[...].astype(jnp.bfloat16)
    h = jnp.dot(x, w1t_ref[...],
                preferred_element_type=jnp.float32) + b1_ref[...]
    g = 0.5 * h * (1.0 + lax.erf(h * 0.7071067811865476))
    logits = jnp.dot(g.astype(jnp.bfloat16), w2t_ref[...],
                     preferred_element_type=jnp.float32) + b2_ref[...]
    logits_ref[...] = logits

    m = jnp.max(logits, axis=-1, keepdims=True)
    e = jnp.exp(logits - m)
    s = jnp.sum(e, axis=-1, keepdims=True)
    w = e / s
    w_ref[...] = w

    iota = lax.broadcasted_iota(jnp.int32, w.shape, 1)
    m1 = jnp.max(w, axis=-1, keepdims=True)
    i1 = jnp.min(jnp.where(w == m1, iota, _E), axis=-1, keepdims=True)
    wm = jnp.where(iota == i1, -1.0, w)
    m2 = jnp.max(wm, axis=-1, keepdims=True)
    i2 = jnp.min(jnp.where(wm == m2, iota, _E), axis=-1, keepdims=True)
    s2 = m1 + m2 + 1e-10
    tkw_ref[...] = jnp.concatenate([m1 / s2, m2 / s2], axis=1)
    tki_ref[...] = jnp.concatenate([i1, i2], axis=1)


def _router_call(instruction_features, w1t, b1r, w2t, b2r):
    B = instruction_features.shape[0]
    grid = (B // _TB,)
    out = pl.pallas_call(
        _router_kernel,
        grid=grid,
        in_specs=[
            pl.BlockSpec((_TB, _HIDDEN), lambda i: (i, 0)),
            pl.BlockSpec((_HIDDEN, _RH), lambda i: (0, 0)),
            pl.BlockSpec((1, _RH), lambda i: (0, 0)),
            pl.BlockSpec((_RH, _E), lambda i: (0, 0)),
            pl.BlockSpec((1, _E), lambda i: (0, 0)),
        ],
        out_specs=[
            pl.BlockSpec((_TB, _E), lambda i: (i, 0)),
            pl.BlockSpec((_TB, 2), lambda i: (i, 0)),
            pl.BlockSpec((_TB, 2), lambda i: (i, 0)),
            pl.BlockSpec((_TB, _E), lambda i: (i, 0)),
        ],
        out_shape=[
            jax.ShapeDtypeStruct((B, _E), jnp.float32),
            jax.ShapeDtypeStruct((B, 2), jnp.float32),
            jax.ShapeDtypeStruct((B, 2), jnp.int32),
            jax.ShapeDtypeStruct((B, _E), jnp.float32),
        ],
    )(instruction_features, w1t, b1r, w2t, b2r)
    routing_weights, top_k_weights, top_k_indices, router_logits = out
    return (routing_weights, top_k_weights, top_k_indices, router_logits)


def kernel(instruction_features, W1, b1, W2, b2):
    w1t = W1.T.astype(jnp.bfloat16)
    w2t = W2.T.astype(jnp.bfloat16)
    b1r = b1.reshape(1, _RH)
    b2r = b2.reshape(1, _E)
    return _router_call(instruction_features, w1t, b1r, w2t, b2r)
